# baseline (speedup 1.0000x reference)
import functools

import jax
import jax.numpy as jnp
from jax.experimental import pallas as pl
from jax.experimental.pallas import tpu as pltpu


def _cdiv(a, b):
    return (a + b - 1) // b


def _recast_body(a_ref, a8_ref, d1_ref, *, bm, np_):
    i = pl.program_id(0)
    a = a_ref[...]
    row = jax.lax.broadcasted_iota(jnp.int32, (bm, np_), 0) + i * bm
    col = jax.lax.broadcasted_iota(jnp.int32, (bm, np_), 1)
    thresh = jnp.where(row == col, 1.5, 0.5).astype(jnp.float32)
    a1 = (a > thresh).astype(jnp.float32)
    a8_ref[...] = a.astype(jnp.int8)
    d = jnp.sum(a1, axis=1)
    dinv = jnp.where(d > 0, jax.lax.rsqrt(d), 0.0)
    d1_ref[...] = jnp.broadcast_to(dinv[:, None], (bm, 8))


def _recast(A, np_, bm=256):
    return pl.pallas_call(
        functools.partial(_recast_body, bm=bm, np_=np_),
        grid=(np_ // bm,),
        in_specs=[pl.BlockSpec((bm, np_), lambda i: (i, 0))],
        out_specs=[
            pl.BlockSpec((bm, np_), lambda i: (i, 0)),
            pl.BlockSpec((bm, 8), lambda i: (i, 0)),
        ],
        out_shape=[
            jax.ShapeDtypeStruct((np_, np_), jnp.int8),
            jax.ShapeDtypeStruct((np_, 8), jnp.float32),
        ],
    )(A)


def _a2_mm_body(lhs_ref, rhs_ref, aij_ref, a2_ref, acc_ref,
                *, bm, bn, nk):
    i = pl.program_id(0)
    j = pl.program_id(1)
    k = pl.program_id(2)
    p = jnp.dot(lhs_ref[...], rhs_ref[...], preferred_element_type=jnp.float32)

    @pl.when(k == 0)
    def _init():
        acc_ref[...] = p

    @pl.when(k > 0)
    def _acc():
        acc_ref[...] += p

    @pl.when(k == nk - 1)
    def _epilogue():
        sb = min(128, bm)
        ns = bm // sb

        def _strip(s, _):
            sl = pl.ds(s * sb, sb)
            row = jax.lax.broadcasted_iota(jnp.int32, (sb, bn), 0) \
                + i * bm + s * sb
            col = jax.lax.broadcasted_iota(jnp.int32, (sb, bn), 1) + j * bn
            thresh = aij_ref[sl, :].astype(jnp.float32) \
                + jnp.where(row == col, 1.5, 0.5)
            a2 = acc_ref[sl, :] > thresh
            a2_ref[sl, :] = a2.astype(jnp.int8)
            return 0

        jax.lax.fori_loop(0, ns, _strip, 0)


def _compute_a2(A8, np_, bm=2048, bn=2048, bk=1024):
    bm, bn, bk = min(bm, np_), min(bn, np_), min(bk, np_)
    ni, nj, nk = np_ // bm, np_ // bn, np_ // bk
    return pl.pallas_call(
        functools.partial(_a2_mm_body, bm=bm, bn=bn, nk=nk),
        grid=(ni, nj, nk),
        in_specs=[
            pl.BlockSpec((bm, bk), lambda i, j, k: (i, k)),
            pl.BlockSpec((bk, bn), lambda i, j, k: (k, j)),
            pl.BlockSpec((bm, bn), lambda i, j, k: (i, j)),
        ],
        out_specs=pl.BlockSpec((bm, bn), lambda i, j, k: (i, j)),
        out_shape=jax.ShapeDtypeStruct((np_, np_), jnp.int8),
        scratch_shapes=[pltpu.VMEM((bm, bn), jnp.float32)],
    )(A8, A8, A8)


def _dinv_body(a_ref, d_ref, *, bm):
    a = a_ref[...].astype(jnp.float32)
    d = jnp.sum(a, axis=1)
    dinv = jnp.where(d > 0, jax.lax.rsqrt(d), 0.0)
    d_ref[...] = jnp.broadcast_to(dinv[:, None], (bm, 8))


def _compute_dinv(a, np_, bm=256):
    return pl.pallas_call(
        functools.partial(_dinv_body, bm=bm),
        grid=(np_ // bm,),
        in_specs=[pl.BlockSpec((bm, np_), lambda i: (i, 0))],
        out_specs=pl.BlockSpec((bm, 8), lambda i: (i, 0)),
        out_shape=jax.ShapeDtypeStruct((np_, 8), jnp.float32),
    )(a)


def _embed_body(x_ref, w_ref, o_ref):
    o_ref[...] = jax.nn.relu(
        jnp.dot(x_ref[...], w_ref[...], preferred_element_type=jnp.float32))


def _embed(x, w, np_, bm=1024):
    f_in, f_out = w.shape
    return pl.pallas_call(
        _embed_body,
        grid=(np_ // bm,),
        in_specs=[
            pl.BlockSpec((bm, f_in), lambda i: (i, 0)),
            pl.BlockSpec((f_in, f_out), lambda i: (0, 0)),
        ],
        out_specs=pl.BlockSpec((bm, f_out), lambda i: (i, 0)),
        out_shape=jax.ShapeDtypeStruct((np_, f_out), jnp.float32),
    )(x, w)


def _prop_body(a8_ref, a2_ref, r_ref, d1k_ref, d2k_ref, d1i_ref, d2i_ref,
               o_ref, acc1_ref, acc2_ref, *, f, bm, bk, nk):
    i = pl.program_id(0)
    k = pl.program_id(1)
    r = r_ref[...]
    u = r * jnp.broadcast_to(d1k_ref[...][:, :1], r.shape)
    v = r * jnp.broadcast_to(d2k_ref[...][:, :1], r.shape)
    row = jax.lax.broadcasted_iota(jnp.int32, (bm, bk), 0) + i * bm
    col = jax.lax.broadcasted_iota(jnp.int32, (bm, bk), 1) + k * bk
    thr = jnp.where(row == col, 1, 0)
    a1 = (a8_ref[...].astype(jnp.int32) > thr).astype(jnp.float32)
    a2 = a2_ref[...].astype(jnp.float32)
    p1 = jnp.dot(a1, u, preferred_element_type=jnp.float32)
    p2 = jnp.dot(a2, v, preferred_element_type=jnp.float32)

    @pl.when(k == 0)
    def _init():
        acc1_ref[...] = p1
        acc2_ref[...] = p2

    @pl.when(k > 0)
    def _acc():
        acc1_ref[...] += p1
        acc2_ref[...] += p2

    @pl.when(k == nk - 1)
    def _fin():
        r1 = jax.nn.relu(acc1_ref[...] * jnp.broadcast_to(d1i_ref[...][:, :1], (acc1_ref.shape)))
        r2 = jax.nn.relu(acc2_ref[...] * jnp.broadcast_to(d2i_ref[...][:, :1], (acc2_ref.shape)))
        o_ref[...] = jnp.concatenate([r1, r2], axis=1)


def _propagate(a8, a2, r, D1, D2, np_, bm=1024, bk=1024):
    f = r.shape[1]
    ni, nk = np_ // bm, np_ // bk
    return pl.pallas_call(
        functools.partial(_prop_body, f=f, bm=bm, bk=bk, nk=nk),
        grid=(ni, nk),
        in_specs=[
            pl.BlockSpec((bm, bk), lambda i, k: (i, k)),
            pl.BlockSpec((bm, bk), lambda i, k: (i, k)),
            pl.BlockSpec((bk, f), lambda i, k: (k, 0)),
            pl.BlockSpec((bk, 8), lambda i, k: (k, 0)),
            pl.BlockSpec((bk, 8), lambda i, k: (k, 0)),
            pl.BlockSpec((bm, 8), lambda i, k: (i, 0)),
            pl.BlockSpec((bm, 8), lambda i, k: (i, 0)),
        ],
        out_specs=pl.BlockSpec((bm, 2 * f), lambda i, k: (i, 0)),
        out_shape=jax.ShapeDtypeStruct((np_, 2 * f), jnp.float32),
        scratch_shapes=[
            pltpu.VMEM((bm, f), jnp.float32),
            pltpu.VMEM((bm, f), jnp.float32),
        ],
    )(a8, a2, r, D1, D2, D1, D2)


def _cls_body(r0_ref, r1_ref, r2_ref, w_ref, o_ref):
    rf = jnp.concatenate([r0_ref[...], r1_ref[...], r2_ref[...]], axis=1)
    l = jnp.dot(rf, w_ref[...], preferred_element_type=jnp.float32)
    m = jnp.max(l, axis=1, keepdims=True)
    s = jnp.log(jnp.sum(jnp.exp(l - m), axis=1, keepdims=True))
    o_ref[...] = l - m - s


def _classify(r0, r1, r2, w, np_, bm=512):
    f0, f1, f2 = r0.shape[1], r1.shape[1], r2.shape[1]
    ftot, c = w.shape
    return pl.pallas_call(
        _cls_body,
        grid=(np_ // bm,),
        in_specs=[
            pl.BlockSpec((bm, f0), lambda i: (i, 0)),
            pl.BlockSpec((bm, f1), lambda i: (i, 0)),
            pl.BlockSpec((bm, f2), lambda i: (i, 0)),
            pl.BlockSpec((ftot, c), lambda i: (0, 0)),
        ],
        out_specs=pl.BlockSpec((bm, c), lambda i: (i, 0)),
        out_shape=jax.ShapeDtypeStruct((np_, c), jnp.float32),
    )(r0, r1, r2, w)


def kernel(x, edge_index, w_embed, w_classify):
    n = x.shape[0]
    np_ = _cdiv(n, 1024) * 1024

    A = jnp.zeros((np_, np_), jnp.float32)
    A = A.at[edge_index[0], edge_index[1]].add(1.0)

    A8, D1 = _recast(A, np_)
    a2 = _compute_a2(A8, np_)
    D2 = _compute_dinv(a2, np_)

    xp = jnp.pad(x, ((0, np_ - n), (0, 0)))
    r0 = _embed(xp, w_embed, np_)
    r1 = _propagate(A8, a2, r0, D1, D2, np_)
    r2 = _propagate(A8, a2, r1, D1, D2, np_)
    out = _classify(r0, r1, r2, w_classify, np_)
    return out[:n]


# speedup vs baseline: 1.1264x; 1.1264x over previous
import functools

import jax
import jax.numpy as jnp
from jax.experimental import pallas as pl
from jax.experimental.pallas import tpu as pltpu


def _cdiv(a, b):
    return (a + b - 1) // b


def _recast_body(a_ref, a8_ref, d1_ref, *, bm, np_):
    i = pl.program_id(0)
    a = a_ref[...]
    row = jax.lax.broadcasted_iota(jnp.int32, (bm, np_), 0) + i * bm
    col = jax.lax.broadcasted_iota(jnp.int32, (bm, np_), 1)
    thresh = jnp.where(row == col, 1.5, 0.5).astype(jnp.float32)
    a1 = (a > thresh).astype(jnp.float32)
    a8_ref[...] = a.astype(jnp.int8)
    d = jnp.sum(a1, axis=1)
    dinv = jnp.where(d > 0, jax.lax.rsqrt(d), 0.0)
    d1_ref[...] = jnp.broadcast_to(dinv[:, None], (bm, 8))


def _recast(A, np_, bm=256):
    return pl.pallas_call(
        functools.partial(_recast_body, bm=bm, np_=np_),
        grid=(np_ // bm,),
        in_specs=[pl.BlockSpec((bm, np_), lambda i: (i, 0))],
        out_specs=[
            pl.BlockSpec((bm, np_), lambda i: (i, 0)),
            pl.BlockSpec((bm, 8), lambda i: (i, 0)),
        ],
        out_shape=[
            jax.ShapeDtypeStruct((np_, np_), jnp.int8),
            jax.ShapeDtypeStruct((np_, 8), jnp.float32),
        ],
    )(A)


def _a2_mm_body(lhs_ref, rhs_ref, aij_ref, a2_ref, *, bm, bn):
    i = pl.program_id(0)
    j = pl.program_id(1)
    p = jnp.dot(lhs_ref[...], rhs_ref[...], preferred_element_type=jnp.float32)
    row = jax.lax.broadcasted_iota(jnp.int32, (bm, bn), 0) + i * bm
    col = jax.lax.broadcasted_iota(jnp.int32, (bm, bn), 1) + j * bn
    thresh = aij_ref[...].astype(jnp.float32) + jnp.where(row == col, 1.5, 0.5)
    a2_ref[...] = (p > thresh).astype(jnp.int8)


def _compute_a2(A8, np_, bm=256, bn=2048):
    bm, bn = min(bm, np_), min(bn, np_)
    ni, nj = np_ // bm, np_ // bn
    return pl.pallas_call(
        functools.partial(_a2_mm_body, bm=bm, bn=bn),
        grid=(nj, ni),
        in_specs=[
            pl.BlockSpec((bm, np_), lambda j, i: (i, 0)),
            pl.BlockSpec((np_, bn), lambda j, i: (0, j)),
            pl.BlockSpec((bm, bn), lambda j, i: (i, j)),
        ],
        out_specs=pl.BlockSpec((bm, bn), lambda j, i: (i, j)),
        out_shape=jax.ShapeDtypeStruct((np_, np_), jnp.int8),
    )(A8, A8, A8)


def _dinv_body(a_ref, d_ref, *, bm):
    a = a_ref[...].astype(jnp.float32)
    d = jnp.sum(a, axis=1)
    dinv = jnp.where(d > 0, jax.lax.rsqrt(d), 0.0)
    d_ref[...] = jnp.broadcast_to(dinv[:, None], (bm, 8))


def _compute_dinv(a, np_, bm=256):
    return pl.pallas_call(
        functools.partial(_dinv_body, bm=bm),
        grid=(np_ // bm,),
        in_specs=[pl.BlockSpec((bm, np_), lambda i: (i, 0))],
        out_specs=pl.BlockSpec((bm, 8), lambda i: (i, 0)),
        out_shape=jax.ShapeDtypeStruct((np_, 8), jnp.float32),
    )(a)


def _embed_body(x_ref, w_ref, o_ref):
    o_ref[...] = jax.nn.relu(
        jnp.dot(x_ref[...], w_ref[...], preferred_element_type=jnp.float32))


def _embed(x, w, np_, bm=1024):
    f_in, f_out = w.shape
    return pl.pallas_call(
        _embed_body,
        grid=(np_ // bm,),
        in_specs=[
            pl.BlockSpec((bm, f_in), lambda i: (i, 0)),
            pl.BlockSpec((f_in, f_out), lambda i: (0, 0)),
        ],
        out_specs=pl.BlockSpec((bm, f_out), lambda i: (i, 0)),
        out_shape=jax.ShapeDtypeStruct((np_, f_out), jnp.float32),
    )(x, w)


def _prop_body(a8_ref, a2_ref, r_ref, d1k_ref, d2k_ref, d1i_ref, d2i_ref,
               o_ref, acc1_ref, acc2_ref, *, f, bm, bk, nk):
    i = pl.program_id(0)
    k = pl.program_id(1)
    r = r_ref[...]
    u = r * jnp.broadcast_to(d1k_ref[...][:, :1], r.shape)
    v = r * jnp.broadcast_to(d2k_ref[...][:, :1], r.shape)
    row = jax.lax.broadcasted_iota(jnp.int32, (bm, bk), 0) + i * bm
    col = jax.lax.broadcasted_iota(jnp.int32, (bm, bk), 1) + k * bk
    thr = jnp.where(row == col, 1, 0)
    a1 = (a8_ref[...].astype(jnp.int32) > thr).astype(jnp.float32)
    a2 = a2_ref[...].astype(jnp.float32)
    p1 = jnp.dot(a1, u, preferred_element_type=jnp.float32)
    p2 = jnp.dot(a2, v, preferred_element_type=jnp.float32)

    @pl.when(k == 0)
    def _init():
        acc1_ref[...] = p1
        acc2_ref[...] = p2

    @pl.when(k > 0)
    def _acc():
        acc1_ref[...] += p1
        acc2_ref[...] += p2

    @pl.when(k == nk - 1)
    def _fin():
        r1 = jax.nn.relu(acc1_ref[...] * jnp.broadcast_to(d1i_ref[...][:, :1], (acc1_ref.shape)))
        r2 = jax.nn.relu(acc2_ref[...] * jnp.broadcast_to(d2i_ref[...][:, :1], (acc2_ref.shape)))
        o_ref[...] = jnp.concatenate([r1, r2], axis=1)


def _propagate(a8, a2, r, D1, D2, np_, bm=1024, bk=1024):
    f = r.shape[1]
    ni, nk = np_ // bm, np_ // bk
    return pl.pallas_call(
        functools.partial(_prop_body, f=f, bm=bm, bk=bk, nk=nk),
        grid=(ni, nk),
        in_specs=[
            pl.BlockSpec((bm, bk), lambda i, k: (i, k)),
            pl.BlockSpec((bm, bk), lambda i, k: (i, k)),
            pl.BlockSpec((bk, f), lambda i, k: (k, 0)),
            pl.BlockSpec((bk, 8), lambda i, k: (k, 0)),
            pl.BlockSpec((bk, 8), lambda i, k: (k, 0)),
            pl.BlockSpec((bm, 8), lambda i, k: (i, 0)),
            pl.BlockSpec((bm, 8), lambda i, k: (i, 0)),
        ],
        out_specs=pl.BlockSpec((bm, 2 * f), lambda i, k: (i, 0)),
        out_shape=jax.ShapeDtypeStruct((np_, 2 * f), jnp.float32),
        scratch_shapes=[
            pltpu.VMEM((bm, f), jnp.float32),
            pltpu.VMEM((bm, f), jnp.float32),
        ],
    )(a8, a2, r, D1, D2, D1, D2)


def _cls_body(r0_ref, r1_ref, r2_ref, w_ref, o_ref):
    rf = jnp.concatenate([r0_ref[...], r1_ref[...], r2_ref[...]], axis=1)
    l = jnp.dot(rf, w_ref[...], preferred_element_type=jnp.float32)
    m = jnp.max(l, axis=1, keepdims=True)
    s = jnp.log(jnp.sum(jnp.exp(l - m), axis=1, keepdims=True))
    o_ref[...] = l - m - s


def _classify(r0, r1, r2, w, np_, bm=512):
    f0, f1, f2 = r0.shape[1], r1.shape[1], r2.shape[1]
    ftot, c = w.shape
    return pl.pallas_call(
        _cls_body,
        grid=(np_ // bm,),
        in_specs=[
            pl.BlockSpec((bm, f0), lambda i: (i, 0)),
            pl.BlockSpec((bm, f1), lambda i: (i, 0)),
            pl.BlockSpec((bm, f2), lambda i: (i, 0)),
            pl.BlockSpec((ftot, c), lambda i: (0, 0)),
        ],
        out_specs=pl.BlockSpec((bm, c), lambda i: (i, 0)),
        out_shape=jax.ShapeDtypeStruct((np_, c), jnp.float32),
    )(r0, r1, r2, w)


def kernel(x, edge_index, w_embed, w_classify):
    n = x.shape[0]
    np_ = _cdiv(n, 1024) * 1024

    A = jnp.zeros((np_, np_), jnp.float32)
    A = A.at[edge_index[0], edge_index[1]].add(1.0)

    A8, D1 = _recast(A, np_)
    a2 = _compute_a2(A8, np_)
    D2 = _compute_dinv(a2, np_)

    xp = jnp.pad(x, ((0, np_ - n), (0, 0)))
    r0 = _embed(xp, w_embed, np_)
    r1 = _propagate(A8, a2, r0, D1, D2, np_)
    r2 = _propagate(A8, a2, r1, D1, D2, np_)
    out = _classify(r0, r1, r2, w_classify, np_)
    return out[:n]
